# Initial kernel scaffold; baseline (speedup 1.0000x reference)
import jax
import jax.numpy as jnp
from jax.experimental import pallas as pl


def kernel(hidden_states, router_weight, q_lora_a, q_lora_b, v_lora_a, v_lora_b):
    raise NotImplementedError("write your pallas kernel here")


# speedup vs baseline: 10.9955x; 10.9955x over previous
import functools

import jax
import jax.numpy as jnp
from jax.experimental import pallas as pl
from jax.experimental.pallas import tpu as pltpu

E = 8
RANK = 32
D = 2048
ALPHA = 32.0
BM = 256


def _body(h_ref, wt_ref, aq_ref, bq_ref, av_ref, bv_ref, q_ref, v_ref):
    h = h_ref[...]
    logits = jnp.dot(h, wt_ref[...], preferred_element_type=jnp.float32)
    m = jnp.max(logits, axis=1, keepdims=True)
    p = jnp.exp(logits - m)
    score = 1.0 / jnp.sum(p, axis=1, keepdims=True)
    idx = jnp.argmax(logits, axis=1)
    s = score * (ALPHA / float(RANK))

    col_expert = jax.lax.broadcasted_iota(jnp.int32, (BM, E * RANK), 1) // RANK
    keep = col_expert == idx[:, None]

    tq = jnp.dot(h, aq_ref[...], preferred_element_type=jnp.float32)
    tq = jnp.where(keep, tq, 0.0)
    q_ref[...] = jnp.dot(tq, bq_ref[...], preferred_element_type=jnp.float32) * s

    tv = jnp.dot(h, av_ref[...], preferred_element_type=jnp.float32)
    tv = jnp.where(keep, tv, 0.0)
    v_ref[...] = jnp.dot(tv, bv_ref[...], preferred_element_type=jnp.float32) * s


@jax.jit
def _run(h, wt, aq, bq, av, bv):
    n_tokens = h.shape[0]
    grid = (n_tokens // BM,)
    full = lambda shape: pl.BlockSpec(shape, lambda i: (0, 0))
    q, v = pl.pallas_call(
        _body,
        grid=grid,
        in_specs=[
            pl.BlockSpec((BM, D), lambda i: (i, 0)),
            full((D, E)),
            full((D, E * RANK)),
            full((E * RANK, D)),
            full((D, E * RANK)),
            full((E * RANK, D)),
        ],
        out_specs=[
            pl.BlockSpec((BM, D), lambda i: (i, 0)),
            pl.BlockSpec((BM, D), lambda i: (i, 0)),
        ],
        out_shape=[
            jax.ShapeDtypeStruct((n_tokens, D), jnp.float32),
            jax.ShapeDtypeStruct((n_tokens, D), jnp.float32),
        ],
    )(h, wt, aq, bq, av, bv)
    return q, v


def kernel(hidden_states, router_weight, q_lora_a, q_lora_b, v_lora_a, v_lora_b):
    orig_shape = hidden_states.shape[:-1]
    h = hidden_states.reshape(-1, hidden_states.shape[-1])
    wt = router_weight.T
    aq = q_lora_a.transpose(1, 0, 2).reshape(D, E * RANK)
    bq = q_lora_b.reshape(E * RANK, -1)
    av = v_lora_a.transpose(1, 0, 2).reshape(D, E * RANK)
    bv = v_lora_b.reshape(E * RANK, -1)
    q, v = _run(h, wt, aq, bq, av, bv)
    q_out = q_lora_b.shape[-1]
    v_out = v_lora_b.shape[-1]
    return (q.reshape(*orig_shape, q_out), v.reshape(*orig_shape, v_out))


# speedup vs baseline: 12.4703x; 1.1341x over previous
import functools

import jax
import jax.numpy as jnp
from jax.experimental import pallas as pl
from jax.experimental.pallas import tpu as pltpu

E = 8
RANK = 32
D = 2048
ALPHA = 32.0
BM = 512


def _body(h_ref, wt_ref, aq_ref, bq_ref, av_ref, bv_ref, q_ref, v_ref):
    h = h_ref[...]
    logits = jnp.dot(h, wt_ref[...], preferred_element_type=jnp.float32)
    m = jnp.max(logits, axis=1, keepdims=True)
    p = jnp.exp(logits - m)
    score = 1.0 / jnp.sum(p, axis=1, keepdims=True)
    idx = jnp.argmax(logits, axis=1)
    s = score * (ALPHA / float(RANK))

    col_expert = jax.lax.broadcasted_iota(jnp.int32, (BM, E * RANK), 1) // RANK
    keep = col_expert == idx[:, None]

    tq = jnp.dot(h, aq_ref[...], preferred_element_type=jnp.float32)
    tq = jnp.where(keep, tq, 0.0)
    q_ref[...] = jnp.dot(tq, bq_ref[...], preferred_element_type=jnp.float32) * s

    tv = jnp.dot(h, av_ref[...], preferred_element_type=jnp.float32)
    tv = jnp.where(keep, tv, 0.0)
    v_ref[...] = jnp.dot(tv, bv_ref[...], preferred_element_type=jnp.float32) * s


@jax.jit
def _run(h, wt, aq, bq, av, bv):
    n_tokens = h.shape[0]
    grid = (n_tokens // BM,)
    full = lambda shape: pl.BlockSpec(shape, lambda i: (0, 0))
    q, v = pl.pallas_call(
        _body,
        grid=grid,
        in_specs=[
            pl.BlockSpec((BM, D), lambda i: (i, 0)),
            full((D, E)),
            full((D, E * RANK)),
            full((E * RANK, D)),
            full((D, E * RANK)),
            full((E * RANK, D)),
        ],
        out_specs=[
            pl.BlockSpec((BM, D), lambda i: (i, 0)),
            pl.BlockSpec((BM, D), lambda i: (i, 0)),
        ],
        out_shape=[
            jax.ShapeDtypeStruct((n_tokens, D), jnp.float32),
            jax.ShapeDtypeStruct((n_tokens, D), jnp.float32),
        ],
    )(h, wt, aq, bq, av, bv)
    return q, v


def kernel(hidden_states, router_weight, q_lora_a, q_lora_b, v_lora_a, v_lora_b):
    orig_shape = hidden_states.shape[:-1]
    h = hidden_states.reshape(-1, hidden_states.shape[-1])
    wt = router_weight.T
    aq = q_lora_a.transpose(1, 0, 2).reshape(D, E * RANK)
    bq = q_lora_b.reshape(E * RANK, -1)
    av = v_lora_a.transpose(1, 0, 2).reshape(D, E * RANK)
    bv = v_lora_b.reshape(E * RANK, -1)
    q, v = _run(h, wt, aq, bq, av, bv)
    q_out = q_lora_b.shape[-1]
    v_out = v_lora_b.shape[-1]
    return (q.reshape(*orig_shape, q_out), v.reshape(*orig_shape, v_out))
